# initial kernel scaffold (unmeasured)
import jax
import jax.numpy as jnp
from jax import lax
from jax.experimental import pallas as pl
from jax.experimental.pallas import tpu as pltpu

N_DEV = 16


def kernel(x, w_mat):
    m_tot, k_loc = x.shape
    n = w_mat.shape[1]
    blk = m_tot // N_DEV

    x3 = x.astype(jnp.bfloat16).reshape(N_DEV, blk, k_loc)
    w_bf = w_mat.astype(jnp.bfloat16)

    def body(x_ref, w_ref, out_ref, xg_ref, send_sems, recv_sems):
        my = lax.axis_index("i")

        xg_ref[pl.ds(my, 1)] = x_ref[pl.ds(my, 1)]

        for j in range(N_DEV):
            @pl.when(my != j)
            def _(j=j):
                pltpu.make_async_remote_copy(
                    src_ref=x_ref.at[pl.ds(j, 1)],
                    dst_ref=xg_ref.at[pl.ds(my, 1)],
                    send_sem=send_sems.at[j],
                    recv_sem=recv_sems.at[my],
                    device_id=(j,),
                    device_id_type=pl.DeviceIdType.MESH,
                ).start()

        for j in range(N_DEV):
            @pl.when(my != j)
            def _(j=j):
                pltpu.make_async_remote_copy(
                    src_ref=x_ref.at[pl.ds(j, 1)],
                    dst_ref=xg_ref.at[pl.ds(j, 1)],
                    send_sem=send_sems.at[j],
                    recv_sem=recv_sems.at[j],
                    device_id=(j,),
                    device_id_type=pl.DeviceIdType.MESH,
                ).wait_recv()

        acc = jnp.zeros((blk, n), jnp.float32)
        for j in range(N_DEV):
            acc = acc + jnp.dot(
                xg_ref[j],
                w_ref[j * blk:(j + 1) * blk, :],
                preferred_element_type=jnp.float32,
            )
        c = 0.7978845608028654
        out_ref[...] = 0.5 * acc * (1.0 + jnp.tanh(c * (acc + 0.044715 * acc ** 3)))

        for j in range(N_DEV):
            @pl.when(my != j)
            def _(j=j):
                pltpu.make_async_remote_copy(
                    src_ref=x_ref.at[pl.ds(j, 1)],
                    dst_ref=xg_ref.at[pl.ds(j, 1)],
                    send_sem=send_sems.at[j],
                    recv_sem=recv_sems.at[j],
                    device_id=(j,),
                    device_id_type=pl.DeviceIdType.MESH,
                ).wait_send()

    return pl.pallas_call(
        body,
        out_shape=jax.ShapeDtypeStruct((blk, n), jnp.float32),
        in_specs=[
            pl.BlockSpec(memory_space=pltpu.VMEM),
            pl.BlockSpec(memory_space=pltpu.VMEM),
        ],
        out_specs=pl.BlockSpec(memory_space=pltpu.VMEM),
        scratch_shapes=[
            pltpu.VMEM((N_DEV, blk, k_loc), jnp.bfloat16),
            pltpu.SemaphoreType.DMA((N_DEV,)),
            pltpu.SemaphoreType.DMA((N_DEV,)),
        ],
        compiler_params=pltpu.CompilerParams(collective_id=0),
    )(x3, w_bf)


# baseline (device time: 19498 ns/iter reference)
import jax
import jax.numpy as jnp
from jax import lax
from jax.experimental import pallas as pl
from jax.experimental.pallas import tpu as pltpu

N_DEV = 16


def kernel(x, w_mat):
    m_tot, k_loc = x.shape
    n = w_mat.shape[1]
    blk = m_tot // N_DEV

    x3 = x.astype(jnp.bfloat16).reshape(N_DEV, blk, k_loc)
    w_bf = w_mat.astype(jnp.bfloat16)

    def body(x_ref, w_ref, out_ref, xg_ref, send_sems, recv_sems):
        my = lax.axis_index("i")

        xg_ref[pl.ds(my, 1)] = x_ref[pl.ds(my, 1)]

        for j in range(N_DEV):
            @pl.when(my != j)
            def _(j=j):
                pltpu.make_async_remote_copy(
                    src_ref=x_ref.at[pl.ds(j, 1)],
                    dst_ref=xg_ref.at[pl.ds(my, 1)],
                    send_sem=send_sems.at[j],
                    recv_sem=recv_sems.at[my],
                    device_id=(j,),
                    device_id_type=pl.DeviceIdType.MESH,
                ).start()

        for j in range(N_DEV):
            @pl.when(my != j)
            def _(j=j):
                pltpu.make_async_remote_copy(
                    src_ref=x_ref.at[pl.ds(j, 1)],
                    dst_ref=xg_ref.at[pl.ds(j, 1)],
                    send_sem=send_sems.at[j],
                    recv_sem=recv_sems.at[j],
                    device_id=(j,),
                    device_id_type=pl.DeviceIdType.MESH,
                ).wait_recv()

        acc = jnp.zeros((blk, n), jnp.float32)
        for j in range(N_DEV):
            acc = acc + jnp.dot(
                xg_ref[j],
                w_ref[j * blk:(j + 1) * blk, :],
                preferred_element_type=jnp.float32,
            )
        c = 0.7978845608028654
        out_ref[...] = 0.5 * acc * (1.0 + jnp.tanh(c * (acc + 0.044715 * acc ** 3)))

        for j in range(N_DEV):
            @pl.when(my != j)
            def _(j=j):
                pltpu.make_async_remote_copy(
                    src_ref=x_ref.at[pl.ds(j, 1)],
                    dst_ref=xg_ref.at[pl.ds(j, 1)],
                    send_sem=send_sems.at[j],
                    recv_sem=recv_sems.at[j],
                    device_id=(j,),
                    device_id_type=pl.DeviceIdType.MESH,
                ).wait_send()

    return pl.pallas_call(
        body,
        out_shape=jax.ShapeDtypeStruct((blk, n), jnp.float32),
        in_specs=[
            pl.BlockSpec(memory_space=pltpu.VMEM),
            pl.BlockSpec(memory_space=pltpu.VMEM),
        ],
        out_specs=pl.BlockSpec(memory_space=pltpu.VMEM),
        scratch_shapes=[
            pltpu.VMEM((N_DEV, blk, k_loc), jnp.bfloat16),
            pltpu.SemaphoreType.DMA((N_DEV,)),
            pltpu.SemaphoreType.DMA((N_DEV,)),
        ],
    )(x3, w_bf)


# device time: 14894 ns/iter; 1.3091x vs baseline; 1.3091x over previous
import jax
import jax.numpy as jnp
from jax import lax
from jax.experimental import pallas as pl
from jax.experimental.pallas import tpu as pltpu

N_DEV = 16


def kernel(x, w_mat):
    m_tot, k_loc = x.shape
    n = w_mat.shape[1]
    blk = m_tot // N_DEV

    x3 = x.reshape(N_DEV, blk, k_loc)

    def body(x_ref, w_hbm, out_ref, xl_ref, w32_ref, xg_ref,
             w_sem, send_sems, recv_sems):
        my = lax.axis_index("i")

        w_cp = pltpu.make_async_copy(w_hbm, w32_ref, w_sem)
        w_cp.start()

        xl_ref[...] = x_ref[...].astype(jnp.bfloat16)

        barrier_sem = pltpu.get_barrier_semaphore()
        for j in range(N_DEV):
            @pl.when(my != j)
            def _(j=j):
                pl.semaphore_signal(
                    barrier_sem, inc=1,
                    device_id=(j,), device_id_type=pl.DeviceIdType.MESH,
                )
        pl.semaphore_wait(barrier_sem, N_DEV - 1)

        xg_ref[pl.ds(my, 1)] = xl_ref[pl.ds(my, 1)]

        for j in range(N_DEV):
            @pl.when(my != j)
            def _(j=j):
                pltpu.make_async_remote_copy(
                    src_ref=xl_ref.at[pl.ds(j, 1)],
                    dst_ref=xg_ref.at[pl.ds(my, 1)],
                    send_sem=send_sems.at[j],
                    recv_sem=recv_sems.at[my],
                    device_id=(j,),
                    device_id_type=pl.DeviceIdType.MESH,
                ).start()

        for j in range(N_DEV):
            @pl.when(my != j)
            def _(j=j):
                pltpu.make_async_remote_copy(
                    src_ref=xl_ref.at[pl.ds(j, 1)],
                    dst_ref=xg_ref.at[pl.ds(j, 1)],
                    send_sem=send_sems.at[j],
                    recv_sem=recv_sems.at[j],
                    device_id=(j,),
                    device_id_type=pl.DeviceIdType.MESH,
                ).wait_recv()

        w_cp.wait()

        acc = jnp.zeros((blk, n), jnp.float32)
        for j in range(N_DEV):
            acc = acc + jnp.dot(
                xg_ref[j],
                w32_ref[j * blk:(j + 1) * blk, :].astype(jnp.bfloat16),
                preferred_element_type=jnp.float32,
            )
        c = 0.7978845608028654
        out_ref[...] = 0.5 * acc * (1.0 + jnp.tanh(c * (acc + 0.044715 * acc ** 3)))

        for j in range(N_DEV):
            @pl.when(my != j)
            def _(j=j):
                pltpu.make_async_remote_copy(
                    src_ref=xl_ref.at[pl.ds(j, 1)],
                    dst_ref=xg_ref.at[pl.ds(j, 1)],
                    send_sem=send_sems.at[j],
                    recv_sem=recv_sems.at[j],
                    device_id=(j,),
                    device_id_type=pl.DeviceIdType.MESH,
                ).wait_send()

    return pl.pallas_call(
        body,
        out_shape=jax.ShapeDtypeStruct((blk, n), jnp.float32),
        in_specs=[
            pl.BlockSpec(memory_space=pltpu.VMEM),
            pl.BlockSpec(memory_space=pltpu.MemorySpace.HBM),
        ],
        out_specs=pl.BlockSpec(memory_space=pltpu.VMEM),
        scratch_shapes=[
            pltpu.VMEM((N_DEV, blk, k_loc), jnp.bfloat16),
            pltpu.VMEM((m_tot, n), jnp.float32),
            pltpu.VMEM((N_DEV, blk, k_loc), jnp.bfloat16),
            pltpu.SemaphoreType.DMA,
            pltpu.SemaphoreType.DMA((N_DEV,)),
            pltpu.SemaphoreType.DMA((N_DEV,)),
        ],
        compiler_params=pltpu.CompilerParams(collective_id=0),
    )(x3, w_mat)


# device time: 14447 ns/iter; 1.3496x vs baseline; 1.0309x over previous
import jax
import jax.numpy as jnp
from jax import lax
from jax.experimental import pallas as pl
from jax.experimental.pallas import tpu as pltpu

N_DEV = 16


def kernel(x, w_mat):
    m_tot, k_loc = x.shape
    n = w_mat.shape[1]
    blk = m_tot // N_DEV

    def body(x_ref, w_ref, out_ref, xl_ref, wbf_ref, xg_ref,
             send_sems, recv_sems):
        my = lax.axis_index("i")

        xl_ref[...] = x_ref[...].astype(jnp.bfloat16)

        barrier_sem = pltpu.get_barrier_semaphore()
        for j in range(N_DEV):
            @pl.when(my != j)
            def _(j=j):
                pl.semaphore_signal(
                    barrier_sem, inc=1,
                    device_id=(j,), device_id_type=pl.DeviceIdType.MESH,
                )
        pl.semaphore_wait(barrier_sem, N_DEV - 1)

        for d in range(1, N_DEV):
            tgt = lax.rem(my + d, N_DEV)
            pltpu.make_async_remote_copy(
                src_ref=xl_ref.at[pl.ds(tgt * blk, blk), :],
                dst_ref=xg_ref.at[pl.ds(my * blk, blk), :],
                send_sem=send_sems.at[d - 1],
                recv_sem=recv_sems.at[my],
                device_id=(tgt,),
                device_id_type=pl.DeviceIdType.MESH,
            ).start()

        wbf_ref[...] = w_ref[...].astype(jnp.bfloat16)

        acc = jnp.dot(
            xl_ref[pl.ds(my * blk, blk), :],
            wbf_ref[pl.ds(my * blk, blk), :],
            preferred_element_type=jnp.float32,
        )

        for d in range(1, N_DEV):
            src = lax.rem(my - d + N_DEV, N_DEV)
            pltpu.make_async_remote_copy(
                src_ref=xl_ref.at[pl.ds(src * blk, blk), :],
                dst_ref=xg_ref.at[pl.ds(src * blk, blk), :],
                send_sem=send_sems.at[d - 1],
                recv_sem=recv_sems.at[src],
                device_id=(src,),
                device_id_type=pl.DeviceIdType.MESH,
            ).wait_recv()
            acc = acc + jnp.dot(
                xg_ref[pl.ds(src * blk, blk), :],
                wbf_ref[pl.ds(src * blk, blk), :],
                preferred_element_type=jnp.float32,
            )

        c = 0.7978845608028654
        out_ref[...] = 0.5 * acc * (1.0 + jnp.tanh(c * (acc + 0.044715 * acc ** 3)))

        for d in range(1, N_DEV):
            tgt = lax.rem(my + d, N_DEV)
            pltpu.make_async_remote_copy(
                src_ref=xl_ref.at[pl.ds(tgt * blk, blk), :],
                dst_ref=xg_ref.at[pl.ds(my * blk, blk), :],
                send_sem=send_sems.at[d - 1],
                recv_sem=recv_sems.at[my],
                device_id=(tgt,),
                device_id_type=pl.DeviceIdType.MESH,
            ).wait_send()

    return pl.pallas_call(
        body,
        out_shape=jax.ShapeDtypeStruct((blk, n), jnp.float32),
        in_specs=[
            pl.BlockSpec(memory_space=pltpu.VMEM),
            pl.BlockSpec(memory_space=pltpu.VMEM),
        ],
        out_specs=pl.BlockSpec(memory_space=pltpu.VMEM),
        scratch_shapes=[
            pltpu.VMEM((m_tot, k_loc), jnp.bfloat16),
            pltpu.VMEM((m_tot, n), jnp.bfloat16),
            pltpu.VMEM((m_tot, k_loc), jnp.bfloat16),
            pltpu.SemaphoreType.DMA((N_DEV,)),
            pltpu.SemaphoreType.DMA((N_DEV,)),
        ],
        compiler_params=pltpu.CompilerParams(collective_id=0),
    )(x, w_mat)
